# baseline (speedup 1.0000x reference)
import jax
import jax.numpy as jnp
from jax import lax
from jax.experimental import pallas as pl
from jax.experimental.pallas import tpu as pltpu
from jax.experimental.pallas import tpu_sc as plsc

N_CAT = 32
EMB = 64
N_TOTAL = 3 * N_CAT
OUT_D = EMB + N_TOTAL

NC = 2
NS = 16
NW = NC * NS
LANES = 16

BQ = 128
BPL = 1024 // BQ


def _sc_body(v_hbm, r_hbm, s_hbm, p_hbm, t_hbm, out_hbm,
             T_v, r_a, s_a, p_a, v_a, r_b, s_b, p_b, v_b,
             o_0, o_1, o_2, o_3, in_sems, out_sems):
    cid = lax.axis_index("c")
    sid = lax.axis_index("s")
    wid = sid * NC + cid
    n_units = (r_hbm.shape[0] * BPL) // NW
    u0 = wid * n_units

    pltpu.sync_copy(t_hbm, T_v)

    bufs = ((r_a, s_a, p_a, v_a), (r_b, s_b, p_b, v_b))
    obufs = (o_0, o_1, o_2, o_3)

    def in_copies(buf, u):
        rv, sv, pv, vv = bufs[buf]
        ug = u0 + u
        l = ug // BPL
        c0 = (ug % BPL) * BQ
        sem = in_sems.at[buf]
        return (
            pltpu.make_async_copy(r_hbm.at[l, pl.ds(c0, BQ)], rv, sem),
            pltpu.make_async_copy(s_hbm.at[l, pl.ds(c0, BQ)], sv, sem),
            pltpu.make_async_copy(p_hbm.at[l, pl.ds(c0, BQ)], pv, sem),
            pltpu.make_async_copy(
                v_hbm.at[pl.ds(l * EMB, EMB), pl.ds(c0, BQ)], vv, sem),
        )

    def in_start(buf, u):
        for cp in in_copies(buf, u):
            cp.start()

    def in_wait(buf, u):
        for cp in in_copies(buf, u):
            cp.wait()

    def out_copy(ob, u):
        ov = obufs[ob]
        ug = u0 + u
        l = ug // BPL
        c0 = (ug % BPL) * BQ
        return pltpu.make_async_copy(
            ov, out_hbm.at[pl.ds(l * OUT_D, OUT_D), pl.ds(c0, BQ)],
            out_sems.at[ob])

    def compute(buf, ob):
        rv, sv, pv, vv = bufs[buf]
        ov = obufs[ob]

        @plsc.parallel_loop(0, BQ // LANES)
        def _(g):
            go = g * LANES
            r16 = rv[pl.ds(go, LANES)]
            s16 = sv[pl.ds(go, LANES)]
            p16 = pv[pl.ds(go, LANES)]
            sg = s16 + N_CAT
            pg = p16 + 2 * N_CAT

            @plsc.parallel_loop(0, EMB, unroll=4)
            def _(e):
                eb = e * N_TOTAL
                gr = plsc.load_gather(T_v, [r16 + eb])
                gs = plsc.load_gather(T_v, [sg + eb])
                gp = plsc.load_gather(T_v, [pg + eb])
                ov[e, pl.ds(go, LANES)] = \
                    vv[e, pl.ds(go, LANES)] * (gr + gs + gp)

            one = jnp.ones((LANES,), jnp.float32)
            zero = jnp.zeros((LANES,), jnp.float32)

            @plsc.parallel_loop(0, N_CAT, unroll=4)
            def _(c):
                ov[EMB + c, pl.ds(go, LANES)] = \
                    jnp.where(r16 == c, one, zero)
                ov[EMB + N_CAT + c, pl.ds(go, LANES)] = \
                    jnp.where(s16 == c, one, zero)
                ov[EMB + 2 * N_CAT + c, pl.ds(go, LANES)] = \
                    jnp.where(p16 == c, one, zero)

    n4 = n_units // 4
    tail = n_units - n4 * 4
    in_start(0, 0)

    @pl.loop(0, n4)
    def _(i):
        u = 4 * i
        for k in range(4):
            uu = u + k
            nxt = uu + 1

            @pl.when(nxt < n_units)
            def _():
                in_start((k + 1) % 2, nxt)

            in_wait(k % 2, uu)

            @pl.when(i > 0)
            def _():
                out_copy(k, uu - 4).wait()

            compute(k % 2, k)
            out_copy(k, uu).start()

    for k in range(tail):
        uu = n4 * 4 + k
        if uu + 1 < n_units:
            in_start((k + 1) % 2, uu + 1)
        in_wait(k % 2, uu)
        out_copy(uu % 4, uu - 4).wait()
        compute(k % 2, uu % 4)
        out_copy(uu % 4, uu).start()

    for uu in range(n_units - 4, n_units):
        out_copy(uu % 4, uu).wait()


@jax.jit
def _run(vT, rT, sT, pT, Tf):
    L, B = rT.shape
    mesh = plsc.VectorSubcoreMesh(core_axis_name="c", subcore_axis_name="s",
                                  num_cores=NC, num_subcores=NS)
    ibuf = [
        pltpu.VMEM((BQ,), jnp.int32),
        pltpu.VMEM((BQ,), jnp.int32),
        pltpu.VMEM((BQ,), jnp.int32),
        pltpu.VMEM((EMB, BQ), jnp.float32),
    ]
    obuf = [pltpu.VMEM((OUT_D, BQ), jnp.float32)] * 4
    f = pl.kernel(
        _sc_body,
        out_type=jax.ShapeDtypeStruct((L * OUT_D, B), jnp.float32),
        mesh=mesh,
        compiler_params=pltpu.CompilerParams(needs_layout_passes=False),
        scratch_types=[pltpu.VMEM((EMB * N_TOTAL,), jnp.float32)]
        + ibuf + ibuf + obuf
        + [pltpu.SemaphoreType.DMA((2,)), pltpu.SemaphoreType.DMA((4,))],
    )
    return f(vT, rT, sT, pT, Tf)


def kernel(v_t, r_gap, s_gap, p_count, C_w):
    B, L, E = v_t.shape
    vT = jnp.transpose(v_t, (1, 2, 0)).reshape(L * E, B)
    rT = jnp.transpose(r_gap.astype(jnp.int32), (1, 0))
    sT = jnp.transpose(s_gap.astype(jnp.int32), (1, 0))
    pT = jnp.transpose(p_count.astype(jnp.int32), (1, 0))
    Tf = C_w.reshape(EMB * N_TOTAL)
    outT = _run(vT, rT, sT, pT, Tf)
    return jnp.transpose(outT.reshape(L, OUT_D, B), (2, 0, 1))


# speedup vs baseline: 1.0091x; 1.0091x over previous
import jax
import jax.numpy as jnp
from jax import lax
from jax.experimental import pallas as pl
from jax.experimental.pallas import tpu as pltpu
from jax.experimental.pallas import tpu_sc as plsc

N_CAT = 32
EMB = 64
N_TOTAL = 3 * N_CAT
OUT_D = EMB + N_TOTAL

NC = 2
NS = 16
NW = NC * NS
LANES = 16

BQ = 128
BPL = 1024 // BQ


def _sc_body(v_hbm, r_hbm, s_hbm, p_hbm, t_hbm, out_hbm,
             T_v, r_a, s_a, p_a, v_a, o_a, r_b, s_b, p_b, v_b, o_b,
             in_sems, out_sems):
    cid = lax.axis_index("c")
    sid = lax.axis_index("s")
    wid = sid * NC + cid
    n_units = (r_hbm.shape[0] * BPL) // NW
    u0 = wid * n_units
    n2 = n_units // 2

    pltpu.sync_copy(t_hbm, T_v)

    bufs = ((r_a, s_a, p_a, v_a, o_a), (r_b, s_b, p_b, v_b, o_b))

    def in_copies(buf, u):
        rv, sv, pv, vv, _ = bufs[buf]
        ug = u0 + u
        l = ug // BPL
        c0 = (ug % BPL) * BQ
        sem = in_sems.at[buf]
        return (
            pltpu.make_async_copy(r_hbm.at[l, pl.ds(c0, BQ)], rv, sem),
            pltpu.make_async_copy(s_hbm.at[l, pl.ds(c0, BQ)], sv, sem),
            pltpu.make_async_copy(p_hbm.at[l, pl.ds(c0, BQ)], pv, sem),
            pltpu.make_async_copy(
                v_hbm.at[pl.ds(l * EMB, EMB), pl.ds(c0, BQ)], vv, sem),
        )

    def in_start(buf, u):
        for cp in in_copies(buf, u):
            cp.start()

    def in_wait(buf, u):
        for cp in in_copies(buf, u):
            cp.wait()

    def out_copy(buf, u):
        ov = bufs[buf][4]
        ug = u0 + u
        l = ug // BPL
        c0 = (ug % BPL) * BQ
        return pltpu.make_async_copy(
            ov, out_hbm.at[pl.ds(l * OUT_D, OUT_D), pl.ds(c0, BQ)],
            out_sems.at[buf])

    def compute(buf):
        rv, sv, pv, vv, ov = bufs[buf]

        @plsc.parallel_loop(0, BQ // LANES)
        def _(g):
            go = g * LANES
            r16 = rv[pl.ds(go, LANES)]
            s16 = sv[pl.ds(go, LANES)]
            p16 = pv[pl.ds(go, LANES)]
            sg = s16 + N_CAT
            pg = p16 + 2 * N_CAT

            @plsc.parallel_loop(0, EMB, unroll=4)
            def _(e):
                eb = e * N_TOTAL
                gr = plsc.load_gather(T_v, [r16 + eb])
                gs = plsc.load_gather(T_v, [sg + eb])
                gp = plsc.load_gather(T_v, [pg + eb])
                ov[e, pl.ds(go, LANES)] = \
                    vv[e, pl.ds(go, LANES)] * (gr + gs + gp)

            one = jnp.ones((LANES,), jnp.float32)
            zero = jnp.zeros((LANES,), jnp.float32)

            @plsc.parallel_loop(0, N_CAT, unroll=4)
            def _(c):
                ov[EMB + c, pl.ds(go, LANES)] = \
                    jnp.where(r16 == c, one, zero)
                ov[EMB + N_CAT + c, pl.ds(go, LANES)] = \
                    jnp.where(s16 == c, one, zero)
                ov[EMB + 2 * N_CAT + c, pl.ds(go, LANES)] = \
                    jnp.where(p16 == c, one, zero)

    in_start(0, 0)

    @pl.loop(0, n2)
    def _(i):
        c0 = 2 * i
        c1 = c0 + 1
        in_start(1, c1)
        in_wait(0, c0)

        @pl.when(i > 0)
        def _():
            out_copy(0, c0).wait()

        compute(0)
        out_copy(0, c0).start()

        @pl.when(c0 + 2 < n_units)
        def _():
            in_start(0, c0 + 2)

        in_wait(1, c1)

        @pl.when(i > 0)
        def _():
            out_copy(1, c1).wait()

        compute(1)
        out_copy(1, c1).start()

    if n_units % 2:
        u_last = n_units - 1
        in_wait(0, u_last)
        out_copy(0, u_last - 2).wait()
        compute(0)
        out_copy(0, u_last).start()
        out_copy(1, u_last - 1).wait()
        out_copy(0, u_last).wait()
    else:
        out_copy(0, n_units - 2).wait()
        out_copy(1, n_units - 1).wait()


@jax.jit
def _run(vT, rT, sT, pT, Tf):
    L, B = rT.shape
    mesh = plsc.VectorSubcoreMesh(core_axis_name="c", subcore_axis_name="s",
                                  num_cores=NC, num_subcores=NS)
    ibuf = [
        pltpu.VMEM((BQ,), jnp.int32),
        pltpu.VMEM((BQ,), jnp.int32),
        pltpu.VMEM((BQ,), jnp.int32),
        pltpu.VMEM((EMB, BQ), jnp.float32),
        pltpu.VMEM((OUT_D, BQ), jnp.float32),
    ]
    f = pl.kernel(
        _sc_body,
        out_type=jax.ShapeDtypeStruct((L * OUT_D, B), jnp.float32),
        mesh=mesh,
        compiler_params=pltpu.CompilerParams(needs_layout_passes=False),
        scratch_types=[pltpu.VMEM((EMB * N_TOTAL,), jnp.float32)]
        + ibuf + ibuf
        + [pltpu.SemaphoreType.DMA((2,)), pltpu.SemaphoreType.DMA((2,))],
    )
    return f(vT, rT, sT, pT, Tf)


def kernel(v_t, r_gap, s_gap, p_count, C_w):
    B, L, E = v_t.shape
    vT = jnp.transpose(v_t, (1, 2, 0)).reshape(L * E, B)
    rT = jnp.transpose(r_gap.astype(jnp.int32), (1, 0))
    sT = jnp.transpose(s_gap.astype(jnp.int32), (1, 0))
    pT = jnp.transpose(p_count.astype(jnp.int32), (1, 0))
    Tf = C_w.reshape(EMB * N_TOTAL)
    outT = _run(vT, rT, sT, pT, Tf)
    return jnp.transpose(outT.reshape(L, OUT_D, B), (2, 0, 1))
